# baseline (speedup 1.0000x reference)
import functools

import jax
import jax.numpy as jnp
from jax import lax
from jax.experimental import pallas as pl
from jax.experimental.pallas import tpu as pltpu
from jax.experimental.pallas import tpu_sc as plsc

N = 10000
E = 320000
D_IN = 128
HID = 64
HEADS = 8
NCLS = 40

NC = 2
NS = 16
NW = NC * NS
EW = E // NW
B = 128
NCHB = EW // B
PAIRS = NCHB // 2
TAIL = EW - NCHB * B

TW0 = 80
TW1 = 64
RPT = 624
RTAIL = N - NS * RPT


def _leaky(v):
    return jnp.maximum(v, 0.2 * v)


def _vperm(x, idx):
    dn = lax.GatherDimensionNumbers(
        offset_dims=(), collapsed_slice_dims=(0,), start_index_map=(0,))
    return lax.gather(x, idx[:, None], dn, (1,),
                      mode=lax.GatherScatterMode.PROMISE_IN_BOUNDS)


def _tc_layer0(x_ref, w_ref, gs_ref, gd_ref, hsrc_ref, adst_ref):
    h = jnp.dot(x_ref[...], w_ref[...], preferred_element_type=jnp.float32)
    hsrc_ref[...] = jnp.dot(h, gs_ref[...], preferred_element_type=jnp.float32)
    adst_ref[...] = jnp.dot(h, gd_ref[...], preferred_element_type=jnp.float32)


def _tc_mid(acc_ref, r_ref, b0_ref, w1_ref, m1_ref, m2_ref,
            hsrc_ref, adst_ref):
    a = acc_ref[...]
    s = a[0] + a[1]
    den8 = s[:, 0:8]
    num = s[:, 16:80]
    denb = jnp.dot(den8, r_ref[...], preferred_element_type=jnp.float32)
    out0 = num / (denb + 1e-16) + b0_ref[...]
    h1in = jnp.where(out0 > 0.0, out0, jnp.exp(out0) - 1.0)
    h1 = jnp.dot(h1in, w1_ref[...], preferred_element_type=jnp.float32)
    hsrc_ref[...] = jnp.dot(h1, m1_ref[...], preferred_element_type=jnp.float32)
    adst_ref[...] = jnp.dot(h1, m2_ref[...], preferred_element_type=jnp.float32)


def _tc_final(acc_ref, b1_ref, out_ref):
    a = acc_ref[...]
    s = a[0] + a[1]
    den = jnp.broadcast_to(s[:, 0:1], s.shape)
    out_ref[...] = s / (den + 1e-16) + b1_ref[...]


def _tc_call(body, grid, in_specs, out_shapes, out_specs):
    return pl.pallas_call(
        body, grid=(grid,), in_specs=in_specs,
        out_shape=out_shapes, out_specs=out_specs)


def _sc_edge_kernel(tw, head_pairs, broadcast0):
    mesh = plsc.VectorSubcoreMesh(core_axis_name="c", subcore_axis_name="s",
                                  num_cores=NC, num_subcores=NS)

    def buf_types(n):
        return [
            pltpu.VMEM((n, tw), jnp.float32),
            pltpu.VMEM((n, 16), jnp.float32),
            pltpu.VMEM((n, tw), jnp.float32),
            pltpu.VMEM((n,), jnp.int32),
            pltpu.SemaphoreType.DMA,
            pltpu.SemaphoreType.DMA,
        ]

    @functools.partial(
        pl.kernel,
        out_type=jax.ShapeDtypeStruct((NC, N, tw), jnp.float32),
        mesh=mesh,
        scratch_types=(
            [pltpu.VMEM_SHARED((N, tw), jnp.float32),
             pltpu.VMEM((NCHB * B,), jnp.int32),
             pltpu.VMEM((NCHB * B,), jnp.int32),
             pltpu.VMEM((TAIL,), jnp.int32),
             pltpu.VMEM((TAIL,), jnp.int32)]
            + buf_types(B) + buf_types(B) + buf_types(TAIL)),
        compiler_params=pltpu.CompilerParams(use_tc_tiling_on_sc=False,
                                             needs_layout_passes=False),
    )
    def kern(hsrc_hbm, adst_hbm, src_hbm, dst_hbm, zeros_hbm, out_hbm,
             acc, srcall, dstall, tsrc, tdst, *bufs):
        c = lax.axis_index("c")
        s = lax.axis_index("s")
        wid = c * NS + s
        A, Bb, T = bufs[0:6], bufs[6:12], bufs[12:18]

        r0 = s * RPT
        pltpu.sync_copy(zeros_hbm.at[pl.ds(r0, RPT)], acc.at[pl.ds(r0, RPT)])

        @pl.when(s == 0)
        def _():
            pltpu.sync_copy(zeros_hbm.at[pl.ds(NS * RPT, RTAIL)],
                            acc.at[pl.ds(NS * RPT, RTAIL)])

        lane = lax.iota(jnp.int32, 16)
        e0 = wid * EW
        pltpu.sync_copy(src_hbm.at[pl.ds(e0, NCHB * B)], srcall)
        pltpu.sync_copy(dst_hbm.at[pl.ds(e0, NCHB * B)], dstall)
        pltpu.sync_copy(src_hbm.at[pl.ds(e0 + NCHB * B, TAIL)], tsrc)
        pltpu.sync_copy(dst_hbm.at[pl.ds(e0 + NCHB * B, TAIL)], tdst)
        plsc.subcore_barrier()

        def gather_start(buf, ch):
            hr, ar, _, _, sg, _ = buf
            pltpu.make_async_copy(
                hsrc_hbm.at[srcall.at[pl.ds(ch * B, B)]], hr, sg).start()
            pltpu.make_async_copy(
                adst_hbm.at[dstall.at[pl.ds(ch * B, B)]], ar, sg).start()

        def compute_scatter(buf, ch, idsrc, n, tail=False):
            hr, ar, mg, dbuf, sg, ssc = buf
            pltpu.make_async_copy(hsrc_hbm.at[idsrc], hr, sg).wait()
            pltpu.make_async_copy(adst_hbm.at[idsrc], ar, sg).wait()
            if tail:
                dbuf = tdst
            else:
                for t in range(n // 16):
                    dbuf[pl.ds(16 * t, 16)] = dstall[pl.ds(ch * B + 16 * t, 16)]

            def edge_body(e, carry2):
                av = ar[e]
                sv = hr[e, pl.ds(0, 16)]
                ex = jnp.exp(_leaky(av + sv))
                mg[e, pl.ds(0, 16)] = ex
                for j in range(head_pairs):
                    if broadcast0:
                        mult = ex
                    else:
                        mult = _vperm(ex, lane // 8 + 2 * j)
                    hv = hr[e, pl.ds(16 + 16 * j, 16)]
                    mg[e, pl.ds(16 + 16 * j, 16)] = hv * mult
                return carry2

            lax.fori_loop(0, n, edge_body, 0, unroll=4)
            pltpu.make_async_copy(mg, acc.at[dbuf], ssc).start(add=True)

        def scatter_wait(buf, tail=False):
            _, _, mg, dbuf, _, ssc = buf
            if tail:
                dbuf = tdst
            pltpu.make_async_copy(mg, acc.at[dbuf], ssc).wait()

        cdummy = srcall.at[pl.ds(0, B)]

        gather_start(A, 0)
        gather_start(Bb, 1)
        compute_scatter(A, 0, cdummy, B)
        gather_start(A, 2)
        compute_scatter(Bb, 1, cdummy, B)
        gather_start(Bb, 3)

        def pair_body(k, carry):
            scatter_wait(A)
            compute_scatter(A, 2 * k, cdummy, B)
            gather_start(A, 2 * k + 2)
            scatter_wait(Bb)
            compute_scatter(Bb, 2 * k + 1, cdummy, B)
            gather_start(Bb, 2 * k + 3)
            return carry

        lax.fori_loop(1, PAIRS - 1, pair_body, 0)
        hrT, arT, _, _, sgT, _ = T
        pltpu.make_async_copy(hsrc_hbm.at[tsrc], hrT, sgT).start()
        pltpu.make_async_copy(adst_hbm.at[tdst], arT, sgT).start()
        scatter_wait(A)
        compute_scatter(A, 2 * (PAIRS - 1), cdummy, B)
        scatter_wait(Bb)
        compute_scatter(Bb, 2 * (PAIRS - 1) + 1, cdummy, B)
        compute_scatter(T, 0, tsrc, TAIL, tail=True)
        scatter_wait(A)
        scatter_wait(Bb)
        scatter_wait(T, tail=True)

        plsc.subcore_barrier()
        pltpu.sync_copy(acc.at[pl.ds(r0, RPT)], out_hbm.at[c, pl.ds(r0, RPT)])

        @pl.when(s == 0)
        def _():
            pltpu.sync_copy(acc.at[pl.ds(NS * RPT, RTAIL)],
                            out_hbm.at[c, pl.ds(NS * RPT, RTAIL)])

    return kern


_sc_layer0 = _sc_edge_kernel(TW0, 4, False)
_sc_layer1 = _sc_edge_kernel(TW1, 3, True)


def kernel(x, edge_index, W0, a_src0, a_dst0, b0, W1, a_src1, a_dst1, b1):
    f32 = jnp.float32
    src = edge_index[0]
    dst = edge_index[1]

    hh = jnp.arange(HID)
    Asrc0 = jnp.zeros((HID, HEADS), f32).at[hh, hh // 8].set(a_src0.reshape(-1))
    Adst0 = jnp.zeros((HID, HEADS), f32).at[hh, hh // 8].set(a_dst0.reshape(-1))
    E8 = jnp.zeros((HEADS, TW0), f32).at[jnp.arange(8), jnp.arange(8)].set(1.0)
    P0 = jnp.zeros((HID, TW0), f32).at[hh, 16 + hh].set(1.0)
    G0 = Asrc0 @ E8 + P0
    E8_16 = jnp.zeros((HEADS, 16), f32).at[jnp.arange(8), jnp.arange(8)].set(1.0)
    Gd0 = Adst0 @ E8_16

    jj = jnp.arange(NCLS)
    e0 = jnp.zeros((1, TW1), f32).at[0, 0:16].set(1.0)
    P1 = jnp.zeros((NCLS, TW1), f32).at[jj, 16 + jj].set(1.0)
    M1 = a_src1.reshape(NCLS, 1) @ e0 + P1
    e0_16 = jnp.ones((1, 16), f32)
    M2 = a_dst1.reshape(NCLS, 1) @ e0_16

    R = jnp.zeros((HEADS, HID), f32).at[hh // 8, hh].set(1.0)
    b0_row = b0.reshape(1, HID)
    b1_row = jnp.zeros((1, TW1), f32).at[0, 16:16 + NCLS].set(b1)

    z0 = jnp.zeros((N, TW0), f32)
    z1 = jnp.zeros((N, TW1), f32)

    blk = 1000
    g = N // blk
    full = lambda r, c: pl.BlockSpec((r, c), lambda i: (0, 0))
    rows = lambda c: pl.BlockSpec((blk, c), lambda i: (i, 0))
    acc_spec = lambda c: pl.BlockSpec((NC, blk, c), lambda i: (0, i, 0))

    hsrc0, adst0 = _tc_call(
        _tc_layer0, g,
        [rows(D_IN), full(D_IN, HID), full(HID, TW0), full(HID, 16)],
        (jax.ShapeDtypeStruct((N, TW0), f32),
         jax.ShapeDtypeStruct((N, 16), f32)),
        [rows(TW0), rows(16)])(x, W0, G0, Gd0)

    acc0 = _sc_layer0(hsrc0, adst0, src, dst, z0)

    hsrc1, adst1 = _tc_call(
        _tc_mid, g,
        [acc_spec(TW0), full(HEADS, HID), full(1, HID), full(HID, NCLS),
         full(NCLS, TW1), full(NCLS, 16)],
        (jax.ShapeDtypeStruct((N, TW1), f32),
         jax.ShapeDtypeStruct((N, 16), f32)),
        [rows(TW1), rows(16)])(acc0, R, b0_row, W1, M1, M2)

    acc1 = _sc_layer1(hsrc1, adst1, src, dst, z1)

    outf = _tc_call(
        _tc_final, g,
        [acc_spec(TW1), full(1, TW1)],
        jax.ShapeDtypeStruct((N, TW1), f32),
        rows(TW1))(acc1, b1_row)

    return outf[:, 16:16 + NCLS]


# speedup vs baseline: 2.6287x; 2.6287x over previous
import functools

import jax
import jax.numpy as jnp
from jax import lax
from jax.experimental import pallas as pl
from jax.experimental.pallas import tpu as pltpu
from jax.experimental.pallas import tpu_sc as plsc

N = 10000
E = 320000
D_IN = 128
HID = 64
HEADS = 8
NCLS = 40

NC = 2
NS = 16
NW = NC * NS
EW = E // NW
B = 128
NCHB = EW // B
PAIRS = NCHB // 2
TAIL = EW - NCHB * B

TW0 = 80
TW1 = 64
RPT = 624
RTAIL = N - NS * RPT


def _leaky(v):
    return jnp.maximum(v, 0.2 * v)


def _vperm(x, idx):
    dn = lax.GatherDimensionNumbers(
        offset_dims=(), collapsed_slice_dims=(0,), start_index_map=(0,))
    return lax.gather(x, idx[:, None], dn, (1,),
                      mode=lax.GatherScatterMode.PROMISE_IN_BOUNDS)


def _tc_layer0(x_ref, w_ref, gs_ref, gd_ref, hsrc_ref, adst_ref):
    h = jnp.dot(x_ref[...], w_ref[...], preferred_element_type=jnp.float32)
    hsrc_ref[...] = jnp.dot(h, gs_ref[...], preferred_element_type=jnp.float32)
    adst_ref[...] = jnp.dot(h, gd_ref[...], preferred_element_type=jnp.float32)


def _tc_mid(acc_ref, r_ref, b0_ref, w1_ref, m1_ref, m2_ref,
            hsrc_ref, adst_ref):
    a = acc_ref[...]
    s = a[0] + a[1]
    den8 = s[:, 0:8]
    num = s[:, 16:80]
    denb = jnp.dot(den8, r_ref[...], preferred_element_type=jnp.float32)
    out0 = num / (denb + 1e-16) + b0_ref[...]
    h1in = jnp.where(out0 > 0.0, out0, jnp.exp(out0) - 1.0)
    h1 = jnp.dot(h1in, w1_ref[...], preferred_element_type=jnp.float32)
    hsrc_ref[...] = jnp.dot(h1, m1_ref[...], preferred_element_type=jnp.float32)
    adst_ref[...] = jnp.dot(h1, m2_ref[...], preferred_element_type=jnp.float32)


def _tc_final(acc_ref, b1_ref, out_ref):
    a = acc_ref[...]
    s = a[0] + a[1]
    den = jnp.broadcast_to(s[:, 0:1], s.shape)
    out_ref[...] = s / (den + 1e-16) + b1_ref[...]


def _tc_call(body, grid, in_specs, out_shapes, out_specs):
    return pl.pallas_call(
        body, grid=(grid,), in_specs=in_specs,
        out_shape=out_shapes, out_specs=out_specs)


def _sc_edge_kernel(tw, head_pairs, broadcast0):
    mesh = plsc.VectorSubcoreMesh(core_axis_name="c", subcore_axis_name="s",
                                  num_cores=NC, num_subcores=NS)

    def buf_types(n):
        return [
            pltpu.VMEM((n, tw), jnp.float32),
            pltpu.VMEM((n, 16), jnp.float32),
            pltpu.VMEM((n, tw), jnp.float32),
            pltpu.VMEM((n,), jnp.int32),
            pltpu.SemaphoreType.DMA,
            pltpu.SemaphoreType.DMA,
        ]

    @functools.partial(
        pl.kernel,
        out_type=jax.ShapeDtypeStruct((NC, N, tw), jnp.float32),
        mesh=mesh,
        scratch_types=(
            [pltpu.VMEM_SHARED((N, tw), jnp.float32),
             pltpu.VMEM((NCHB * B,), jnp.int32),
             pltpu.VMEM((NCHB * B,), jnp.int32),
             pltpu.VMEM((TAIL,), jnp.int32),
             pltpu.VMEM((TAIL,), jnp.int32)]
            + buf_types(B) + buf_types(B) + buf_types(TAIL)),
        compiler_params=pltpu.CompilerParams(use_tc_tiling_on_sc=False,
                                             needs_layout_passes=False),
    )
    def kern(hsrc_hbm, adst_hbm, src_hbm, dst_hbm, zeros_hbm, out_hbm,
             acc, srcall, dstall, tsrc, tdst, *bufs):
        c = lax.axis_index("c")
        s = lax.axis_index("s")
        wid = c * NS + s
        A, Bb, T = bufs[0:6], bufs[6:12], bufs[12:18]

        r0 = s * RPT
        pltpu.sync_copy(zeros_hbm.at[pl.ds(r0, RPT)], acc.at[pl.ds(r0, RPT)])

        @pl.when(s == 0)
        def _():
            pltpu.sync_copy(zeros_hbm.at[pl.ds(NS * RPT, RTAIL)],
                            acc.at[pl.ds(NS * RPT, RTAIL)])

        lane = lax.iota(jnp.int32, 16)
        e0 = wid * EW
        pltpu.sync_copy(src_hbm.at[pl.ds(e0, NCHB * B)], srcall)
        pltpu.sync_copy(dst_hbm.at[pl.ds(e0, NCHB * B)], dstall)
        pltpu.sync_copy(src_hbm.at[pl.ds(e0 + NCHB * B, TAIL)], tsrc)
        pltpu.sync_copy(dst_hbm.at[pl.ds(e0 + NCHB * B, TAIL)], tdst)
        plsc.subcore_barrier()

        def gather_start(buf, ch):
            hr, ar, _, _, sg, _ = buf
            pltpu.make_async_copy(
                hsrc_hbm.at[srcall.at[pl.ds(ch * B, B)]], hr, sg).start()
            pltpu.make_async_copy(
                adst_hbm.at[dstall.at[pl.ds(ch * B, B)]], ar, sg).start()

        def compute_scatter(buf, ch, idsrc, n, tail=False):
            hr, ar, mg, dbuf, sg, ssc = buf
            pltpu.make_async_copy(hsrc_hbm.at[idsrc], hr, sg).wait()
            pltpu.make_async_copy(adst_hbm.at[idsrc], ar, sg).wait()
            if tail:
                dbuf = tdst
            else:
                for t in range(n // 16):
                    dbuf[pl.ds(16 * t, 16)] = dstall[pl.ds(ch * B + 16 * t, 16)]

            @functools.partial(plsc.parallel_loop, 0, n, unroll=8)
            def edge_body(e):
                av = ar[e]
                sv = hr[e, pl.ds(0, 16)]
                ex = jnp.exp(_leaky(av + sv))
                mg[e, pl.ds(0, 16)] = ex
                for j in range(head_pairs):
                    if broadcast0:
                        mult = ex
                    else:
                        mult = _vperm(ex, lane // 8 + 2 * j)
                    hv = hr[e, pl.ds(16 + 16 * j, 16)]
                    mg[e, pl.ds(16 + 16 * j, 16)] = hv * mult

            pltpu.make_async_copy(mg, acc.at[dbuf], ssc).start(add=True)

        def scatter_wait(buf, tail=False):
            _, _, mg, dbuf, _, ssc = buf
            if tail:
                dbuf = tdst
            pltpu.make_async_copy(mg, acc.at[dbuf], ssc).wait()

        cdummy = srcall.at[pl.ds(0, B)]

        gather_start(A, 0)
        gather_start(Bb, 1)
        compute_scatter(A, 0, cdummy, B)
        gather_start(A, 2)
        compute_scatter(Bb, 1, cdummy, B)
        gather_start(Bb, 3)

        def pair_body(k, carry):
            scatter_wait(A)
            compute_scatter(A, 2 * k, cdummy, B)
            gather_start(A, 2 * k + 2)
            scatter_wait(Bb)
            compute_scatter(Bb, 2 * k + 1, cdummy, B)
            gather_start(Bb, 2 * k + 3)
            return carry

        lax.fori_loop(1, PAIRS - 1, pair_body, 0)
        hrT, arT, _, _, sgT, _ = T
        pltpu.make_async_copy(hsrc_hbm.at[tsrc], hrT, sgT).start()
        pltpu.make_async_copy(adst_hbm.at[tdst], arT, sgT).start()
        scatter_wait(A)
        compute_scatter(A, 2 * (PAIRS - 1), cdummy, B)
        scatter_wait(Bb)
        compute_scatter(Bb, 2 * (PAIRS - 1) + 1, cdummy, B)
        compute_scatter(T, 0, tsrc, TAIL, tail=True)
        scatter_wait(A)
        scatter_wait(Bb)
        scatter_wait(T, tail=True)

        plsc.subcore_barrier()
        pltpu.sync_copy(acc.at[pl.ds(r0, RPT)], out_hbm.at[c, pl.ds(r0, RPT)])

        @pl.when(s == 0)
        def _():
            pltpu.sync_copy(acc.at[pl.ds(NS * RPT, RTAIL)],
                            out_hbm.at[c, pl.ds(NS * RPT, RTAIL)])

    return kern


_sc_layer0 = _sc_edge_kernel(TW0, 4, False)
_sc_layer1 = _sc_edge_kernel(TW1, 3, True)


def kernel(x, edge_index, W0, a_src0, a_dst0, b0, W1, a_src1, a_dst1, b1):
    f32 = jnp.float32
    src = edge_index[0]
    dst = edge_index[1]

    hh = jnp.arange(HID)
    Asrc0 = jnp.zeros((HID, HEADS), f32).at[hh, hh // 8].set(a_src0.reshape(-1))
    Adst0 = jnp.zeros((HID, HEADS), f32).at[hh, hh // 8].set(a_dst0.reshape(-1))
    E8 = jnp.zeros((HEADS, TW0), f32).at[jnp.arange(8), jnp.arange(8)].set(1.0)
    P0 = jnp.zeros((HID, TW0), f32).at[hh, 16 + hh].set(1.0)
    G0 = Asrc0 @ E8 + P0
    E8_16 = jnp.zeros((HEADS, 16), f32).at[jnp.arange(8), jnp.arange(8)].set(1.0)
    Gd0 = Adst0 @ E8_16

    jj = jnp.arange(NCLS)
    e0 = jnp.zeros((1, TW1), f32).at[0, 0:16].set(1.0)
    P1 = jnp.zeros((NCLS, TW1), f32).at[jj, 16 + jj].set(1.0)
    M1 = a_src1.reshape(NCLS, 1) @ e0 + P1
    e0_16 = jnp.ones((1, 16), f32)
    M2 = a_dst1.reshape(NCLS, 1) @ e0_16

    R = jnp.zeros((HEADS, HID), f32).at[hh // 8, hh].set(1.0)
    b0_row = b0.reshape(1, HID)
    b1_row = jnp.zeros((1, TW1), f32).at[0, 16:16 + NCLS].set(b1)

    z0 = jnp.zeros((N, TW0), f32)
    z1 = jnp.zeros((N, TW1), f32)

    blk = 1000
    g = N // blk
    full = lambda r, c: pl.BlockSpec((r, c), lambda i: (0, 0))
    rows = lambda c: pl.BlockSpec((blk, c), lambda i: (i, 0))
    acc_spec = lambda c: pl.BlockSpec((NC, blk, c), lambda i: (0, i, 0))

    hsrc0, adst0 = _tc_call(
        _tc_layer0, g,
        [rows(D_IN), full(D_IN, HID), full(HID, TW0), full(HID, 16)],
        (jax.ShapeDtypeStruct((N, TW0), f32),
         jax.ShapeDtypeStruct((N, 16), f32)),
        [rows(TW0), rows(16)])(x, W0, G0, Gd0)

    acc0 = _sc_layer0(hsrc0, adst0, src, dst, z0)

    hsrc1, adst1 = _tc_call(
        _tc_mid, g,
        [acc_spec(TW0), full(HEADS, HID), full(1, HID), full(HID, NCLS),
         full(NCLS, TW1), full(NCLS, 16)],
        (jax.ShapeDtypeStruct((N, TW1), f32),
         jax.ShapeDtypeStruct((N, 16), f32)),
        [rows(TW1), rows(16)])(acc0, R, b0_row, W1, M1, M2)

    acc1 = _sc_layer1(hsrc1, adst1, src, dst, z1)

    outf = _tc_call(
        _tc_final, g,
        [acc_spec(TW1), full(1, TW1)],
        jax.ShapeDtypeStruct((N, TW1), f32),
        rows(TW1))(acc1, b1_row)

    return outf[:, 16:16 + NCLS]
